# baseline (speedup 1.0000x reference)
import functools

import jax
import jax.numpy as jnp
from jax import lax
from jax.experimental import pallas as pl
from jax.experimental.pallas import tpu as pltpu
from jax.experimental.pallas import tpu_sc as plsc

_NUM_SC = 2
_NUM_SUBCORES = 16
_NUM_WORKERS = _NUM_SC * _NUM_SUBCORES

_TILE = 128
_ALIGN = 8


def _router_plan_body(hs_ref, gw_ref, selb_ref, logits_ref, pos_ref, meta_ref):
    s, e = logits_ref.shape
    hs = hs_ref[...]
    gw = gw_ref[...]
    logits_ref[...] = lax.dot_general(
        hs, gw, (((1,), (1,)), ((), ())), preferred_element_type=jnp.float32)

    sel = selb_ref[...]
    eids = lax.broadcasted_iota(jnp.int32, (s, e), 1)
    oh = (sel == eids).astype(jnp.float32)

    counts_i = jnp.sum(oh, axis=0, keepdims=True).astype(jnp.int32)
    padded_f = jnp.bitwise_and(counts_i + (_ALIGN - 1),
                               -_ALIGN).astype(jnp.float32)
    je = lax.broadcasted_iota(jnp.int32, (e, e), 0)
    ee = lax.broadcasted_iota(jnp.int32, (e, e), 1)
    tri_e = (je < ee).astype(jnp.float32)
    offs_f = lax.dot_general(padded_f, tri_e, (((1,), (0,)), ((), ())),
                             preferred_element_type=jnp.float32)

    ri = lax.broadcasted_iota(jnp.int32, (s, s), 0)
    ci = lax.broadcasted_iota(jnp.int32, (s, s), 1)
    tri_s = (ci < ri).astype(jnp.float32)
    cum_excl = lax.dot_general(tri_s, oh, (((1,), (0,)), ((), ())),
                               preferred_element_type=jnp.float32)
    rank_f = jnp.sum(cum_excl * oh, axis=1, keepdims=True)
    base_f = jnp.sum(offs_f * oh, axis=1, keepdims=True)
    pos = (rank_f + base_f).astype(jnp.int32)
    pos_ref[...] = jnp.broadcast_to(pos, pos_ref.shape)

    meta_ref[0:1, :] = offs_f.astype(jnp.int32)
    meta_ref[1:2, :] = counts_i
    meta_ref[2:8, :] = jnp.zeros((6, e), jnp.int32)


def _expert_ffn_body(off_ref, cnt_ref, xs_ref, w1_ref, w3_ref, w2_ref, ys_ref):
    e = pl.program_id(0)
    f = pl.program_id(1)
    start = off_ref[e]
    n = cnt_ref[e]
    nblk = lax.shift_right_logical(n + (_TILE - 1), 7)
    w1e = w1_ref[0]
    w3e = w3_ref[0]
    w2e = w2_ref[0]

    def body(i, carry):
        s0 = pl.multiple_of(start + i * _TILE, _ALIGN)
        x = xs_ref[pl.ds(s0, _TILE), :]
        h1 = lax.dot_general(x, w1e, (((1,), (1,)), ((), ())),
                             preferred_element_type=jnp.float32)
        h3 = lax.dot_general(x, w3e, (((1,), (1,)), ((), ())),
                             preferred_element_type=jnp.float32)
        h = h1 * (1.0 / (1.0 + jnp.exp(-h1))) * h3
        y = lax.dot_general(h, w2e, (((1,), (1,)), ((), ())),
                            preferred_element_type=jnp.float32)

        @pl.when(f == 0)
        def _init():
            ys_ref[pl.ds(s0, _TILE), :] = y

        @pl.when(f != 0)
        def _acc():
            ys_ref[pl.ds(s0, _TILE), :] += y

        return carry

    lax.fori_loop(0, nblk, body, 0)


def _sc_dispatch_body(hs_hbm, pos_hbm, xs_hbm, idx_v, rows_v, sem):
    tpw = idx_v.shape[0]
    wid = lax.axis_index("s") * _NUM_SC + lax.axis_index("c")
    base = wid * tpw
    pltpu.sync_copy(pos_hbm.at[pl.ds(base, tpw)], idx_v)
    pltpu.sync_copy(hs_hbm.at[pl.ds(base, tpw)], rows_v)
    pltpu.async_copy(rows_v, xs_hbm.at[idx_v], sem).wait()


def _sc_combine_body(ys_hbm, pos_hbm, out_hbm, idx_v, rows_v, sem):
    tpw = idx_v.shape[0]
    wid = lax.axis_index("s") * _NUM_SC + lax.axis_index("c")
    base = wid * tpw
    pltpu.sync_copy(pos_hbm.at[pl.ds(base, tpw)], idx_v)
    pltpu.async_copy(ys_hbm.at[idx_v], rows_v, sem).wait()
    pltpu.sync_copy(rows_v, out_hbm.at[pl.ds(base, tpw)])


def kernel(hidden_states, selected_experts, gate_w, w1, w2, w3):
    b, s, d = hidden_states.shape
    S, H = b * s, d
    E, F, _ = w1.shape
    P = S + _ALIGN * E + _TILE

    hs = hidden_states.reshape(S, H)
    sel = selected_experts.reshape(S, -1)[:, 0].astype(jnp.int32)
    sel_b = jnp.broadcast_to(sel[:, None], (S, E))

    logits, pos8, meta = pl.pallas_call(
        _router_plan_body,
        out_shape=[
            jax.ShapeDtypeStruct((S, E), jnp.float32),
            jax.ShapeDtypeStruct((S, 8), jnp.int32),
            jax.ShapeDtypeStruct((8, E), jnp.int32),
        ],
    )(hs, gate_w, sel_b)
    pos = pos8[:, 0]
    off = meta[0]
    cnt = meta[1]

    mesh = plsc.VectorSubcoreMesh(core_axis_name="c", subcore_axis_name="s")
    tpw = S // _NUM_WORKERS

    dispatch = functools.partial(
        pl.kernel,
        out_type=jax.ShapeDtypeStruct((P, H), jnp.float32),
        mesh=mesh,
        scratch_types=[
            pltpu.VMEM((tpw,), jnp.int32),
            pltpu.VMEM((tpw, H), jnp.float32),
            pltpu.SemaphoreType.DMA,
        ],
    )(_sc_dispatch_body)
    xs = dispatch(hs, pos)

    FS = 2
    grid_spec = pltpu.PrefetchScalarGridSpec(
        num_scalar_prefetch=2,
        grid=(E, FS),
        in_specs=[
            pl.BlockSpec((P, H), lambda e, f, off, cnt: (0, 0)),
            pl.BlockSpec((1, F // FS, H), lambda e, f, off, cnt: (e, f, 0)),
            pl.BlockSpec((1, F // FS, H), lambda e, f, off, cnt: (e, f, 0)),
            pl.BlockSpec((1, H, F // FS), lambda e, f, off, cnt: (e, 0, f)),
        ],
        out_specs=pl.BlockSpec((P, H), lambda e, f, off, cnt: (0, 0)),
    )
    ys = pl.pallas_call(
        _expert_ffn_body,
        grid_spec=grid_spec,
        out_shape=jax.ShapeDtypeStruct((P, H), jnp.float32),
        compiler_params=pltpu.CompilerParams(
            dimension_semantics=("arbitrary", "arbitrary")),
    )(off, cnt, xs, w1, w3, w2)

    combine = functools.partial(
        pl.kernel,
        out_type=jax.ShapeDtypeStruct((S, H), jnp.float32),
        mesh=mesh,
        scratch_types=[
            pltpu.VMEM((tpw,), jnp.int32),
            pltpu.VMEM((tpw, H), jnp.float32),
            pltpu.SemaphoreType.DMA,
        ],
    )(_sc_combine_body)
    final = combine(ys, pos)

    return final.reshape(b, s, d), logits


# speedup vs baseline: 1.1822x; 1.1822x over previous
import functools

import jax
import jax.numpy as jnp
from jax import lax
from jax.experimental import pallas as pl
from jax.experimental.pallas import tpu as pltpu
from jax.experimental.pallas import tpu_sc as plsc

_NUM_SC = 2
_NUM_SUBCORES = 16
_NUM_WORKERS = _NUM_SC * _NUM_SUBCORES

_TILE = 128
_ALIGN = 8


def _router_plan_body(hs_ref, gw_ref, selb_ref, logits_ref, pos_ref, meta_ref):
    s, e = logits_ref.shape
    hs = hs_ref[...]
    gw = gw_ref[...]
    logits_ref[...] = lax.dot_general(
        hs, gw, (((1,), (1,)), ((), ())), preferred_element_type=jnp.float32)

    sel = selb_ref[...]
    eids = lax.broadcasted_iota(jnp.int32, (s, e), 1)
    oh = (sel == eids).astype(jnp.float32)

    counts_i = jnp.sum(oh, axis=0, keepdims=True).astype(jnp.int32)
    padded_f = jnp.bitwise_and(counts_i + (_ALIGN - 1),
                               -_ALIGN).astype(jnp.float32)
    je = lax.broadcasted_iota(jnp.int32, (e, e), 0)
    ee = lax.broadcasted_iota(jnp.int32, (e, e), 1)
    tri_e = (je < ee).astype(jnp.float32)
    offs_f = lax.dot_general(padded_f, tri_e, (((1,), (0,)), ((), ())),
                             preferred_element_type=jnp.float32)

    ri = lax.broadcasted_iota(jnp.int32, (s, s), 0)
    ci = lax.broadcasted_iota(jnp.int32, (s, s), 1)
    tri_s = (ci < ri).astype(jnp.float32)
    cum_excl = lax.dot_general(tri_s, oh, (((1,), (0,)), ((), ())),
                               preferred_element_type=jnp.float32)
    rank_f = jnp.sum(cum_excl * oh, axis=1, keepdims=True)
    base_f = jnp.sum(offs_f * oh, axis=1, keepdims=True)
    pos = (rank_f + base_f).astype(jnp.int32)
    pos_ref[...] = jnp.broadcast_to(pos, pos_ref.shape)

    meta_ref[0:1, :] = offs_f.astype(jnp.int32)
    meta_ref[1:2, :] = counts_i
    meta_ref[2:8, :] = jnp.zeros((6, e), jnp.int32)


_FS = 2
_DEPTH = 4


def _expert_ffn_body(off_ref, cnt_ref, xs_ref, w1_hbm, w3_hbm, w2_hbm, ys_ref,
                     w1b, w3b, w2b, sems):
    fh = w1b.shape[1]
    s = pl.program_id(0)
    nsteps = pl.num_programs(0)

    def copies(step, slot):
        eu = lax.shift_right_logical(step, 1)
        fu = jnp.bitwise_and(step, 1)
        return (
            pltpu.make_async_copy(
                w1_hbm.at[eu, pl.ds(fu * fh, fh), :], w1b.at[slot],
                sems.at[slot, 0]),
            pltpu.make_async_copy(
                w3_hbm.at[eu, pl.ds(fu * fh, fh), :], w3b.at[slot],
                sems.at[slot, 1]),
            pltpu.make_async_copy(
                w2_hbm.at[eu, :, pl.ds(fu * fh, fh)], w2b.at[slot],
                sems.at[slot, 2]),
        )

    @pl.when(s == 0)
    def _prologue():
        for k in range(_DEPTH):
            for c in copies(jnp.int32(k), k):
                c.start()

    slot = lax.rem(s, _DEPTH)
    for c in copies(s, slot):
        c.wait()

    e = lax.shift_right_logical(s, 1)
    f = jnp.bitwise_and(s, 1)
    start = off_ref[e]
    n = cnt_ref[e]
    nblk = lax.shift_right_logical(n + (_TILE - 1), 7)
    w1e = w1b[slot]
    w3e = w3b[slot]
    w2e = w2b[slot]

    def body(i, carry):
        s0 = pl.multiple_of(start + i * _TILE, _ALIGN)
        x = xs_ref[pl.ds(s0, _TILE), :]
        h1 = lax.dot_general(x, w1e, (((1,), (1,)), ((), ())),
                             preferred_element_type=jnp.float32)
        h3 = lax.dot_general(x, w3e, (((1,), (1,)), ((), ())),
                             preferred_element_type=jnp.float32)
        h = h1 * (1.0 / (1.0 + jnp.exp(-h1))) * h3
        y = lax.dot_general(h, w2e, (((1,), (1,)), ((), ())),
                            preferred_element_type=jnp.float32)

        @pl.when(f == 0)
        def _init():
            ys_ref[pl.ds(s0, _TILE), :] = y

        @pl.when(f != 0)
        def _acc():
            ys_ref[pl.ds(s0, _TILE), :] += y

        return carry

    lax.fori_loop(0, nblk, body, 0)

    @pl.when(s + _DEPTH < nsteps)
    def _refill():
        for c in copies(s + _DEPTH, slot):
            c.start()


def _sc_dispatch_body(hs_hbm, pos_hbm, xs_hbm, idx_v, rows_v, sem):
    tpw = idx_v.shape[0]
    wid = lax.axis_index("s") * _NUM_SC + lax.axis_index("c")
    base = wid * tpw
    pltpu.sync_copy(pos_hbm.at[pl.ds(base, tpw)], idx_v)
    pltpu.sync_copy(hs_hbm.at[pl.ds(base, tpw)], rows_v)
    pltpu.async_copy(rows_v, xs_hbm.at[idx_v], sem).wait()


def _sc_combine_body(ys_hbm, pos_hbm, out_hbm, idx_v, rows_v, sem):
    tpw = idx_v.shape[0]
    wid = lax.axis_index("s") * _NUM_SC + lax.axis_index("c")
    base = wid * tpw
    pltpu.sync_copy(pos_hbm.at[pl.ds(base, tpw)], idx_v)
    pltpu.async_copy(ys_hbm.at[idx_v], rows_v, sem).wait()
    pltpu.sync_copy(rows_v, out_hbm.at[pl.ds(base, tpw)])


def kernel(hidden_states, selected_experts, gate_w, w1, w2, w3):
    b, s, d = hidden_states.shape
    S, H = b * s, d
    E, F, _ = w1.shape
    P = S + _ALIGN * E + _TILE

    hs = hidden_states.reshape(S, H)
    sel = selected_experts.reshape(S, -1)[:, 0].astype(jnp.int32)
    sel_b = jnp.broadcast_to(sel[:, None], (S, E))

    logits, pos8, meta = pl.pallas_call(
        _router_plan_body,
        out_shape=[
            jax.ShapeDtypeStruct((S, E), jnp.float32),
            jax.ShapeDtypeStruct((S, 8), jnp.int32),
            jax.ShapeDtypeStruct((8, E), jnp.int32),
        ],
    )(hs, gate_w, sel_b)
    pos = pos8[:, 0]
    off = meta[0]
    cnt = meta[1]

    mesh = plsc.VectorSubcoreMesh(core_axis_name="c", subcore_axis_name="s")
    tpw = S // _NUM_WORKERS

    dispatch = functools.partial(
        pl.kernel,
        out_type=jax.ShapeDtypeStruct((P, H), jnp.float32),
        mesh=mesh,
        scratch_types=[
            pltpu.VMEM((tpw,), jnp.int32),
            pltpu.VMEM((tpw, H), jnp.float32),
            pltpu.SemaphoreType.DMA,
        ],
    )(_sc_dispatch_body)
    xs = dispatch(hs, pos)

    grid_spec = pltpu.PrefetchScalarGridSpec(
        num_scalar_prefetch=2,
        grid=(E * _FS,),
        in_specs=[
            pl.BlockSpec((P, H), lambda s, off, cnt: (0, 0)),
            pl.BlockSpec(memory_space=pltpu.MemorySpace.HBM),
            pl.BlockSpec(memory_space=pltpu.MemorySpace.HBM),
            pl.BlockSpec(memory_space=pltpu.MemorySpace.HBM),
        ],
        out_specs=pl.BlockSpec((P, H), lambda s, off, cnt: (0, 0)),
        scratch_shapes=[
            pltpu.VMEM((_DEPTH, F // _FS, H), jnp.float32),
            pltpu.VMEM((_DEPTH, F // _FS, H), jnp.float32),
            pltpu.VMEM((_DEPTH, H, F // _FS), jnp.float32),
            pltpu.SemaphoreType.DMA((_DEPTH, 3)),
        ],
    )
    ys = pl.pallas_call(
        _expert_ffn_body,
        grid_spec=grid_spec,
        out_shape=jax.ShapeDtypeStruct((P, H), jnp.float32),
        compiler_params=pltpu.CompilerParams(
            dimension_semantics=("arbitrary",)),
    )(off, cnt, xs, w1, w3, w2)

    combine = functools.partial(
        pl.kernel,
        out_type=jax.ShapeDtypeStruct((S, H), jnp.float32),
        mesh=mesh,
        scratch_types=[
            pltpu.VMEM((tpw,), jnp.int32),
            pltpu.VMEM((tpw, H), jnp.float32),
            pltpu.SemaphoreType.DMA,
        ],
    )(_sc_combine_body)
    final = combine(ys, pos)

    return final.reshape(b, s, d), logits
